# baseline (speedup 1.0000x reference)
import functools

import jax
import jax.numpy as jnp
from jax import lax
from jax.experimental import pallas as pl
from jax.experimental.pallas import tpu as pltpu
from jax.experimental.pallas import tpu_sc as plsc

_N = 500000
_D = 128
_NSP = 64

_NC = 2
_NS = 16
_NW = _NC * _NS
_LANES = 16
_NPW = 15616
_TAIL_BASE = _NW * _NPW
_TAIL = _N - _TAIL_BASE


def _sc_gather_body(species_hbm, table_hbm, out_hbm, idx_v, val_v, tab_v):
    wid = lax.axis_index("s") * _NC + lax.axis_index("c")
    base = wid * _NPW
    pltpu.sync_copy(table_hbm, tab_v)
    pltpu.sync_copy(species_hbm.at[pl.ds(base, _NPW)], idx_v)

    def body(i, carry):
        idx = idx_v[pl.ds(i * _LANES, _LANES)]
        val_v[pl.ds(i * _LANES, _LANES)] = plsc.load_gather(tab_v, [idx])
        return carry

    lax.fori_loop(0, _NPW // _LANES, body, 0, unroll=8)
    pltpu.sync_copy(val_v, out_hbm.at[pl.ds(base, _NPW)])

    @pl.when(wid == 0)
    def _tail():
        pltpu.sync_copy(species_hbm.at[pl.ds(_TAIL_BASE, _TAIL)],
                        idx_v.at[pl.ds(0, _TAIL)])

        def tbody(i, carry):
            idx = idx_v[pl.ds(i * _LANES, _LANES)]
            val_v[pl.ds(i * _LANES, _LANES)] = plsc.load_gather(tab_v, [idx])
            return carry

        lax.fori_loop(0, _TAIL // _LANES, tbody, 0, unroll=8)
        pltpu.sync_copy(val_v.at[pl.ds(0, _TAIL)],
                        out_hbm.at[pl.ds(_TAIL_BASE, _TAIL)])


@functools.cache
def _sc_gather():
    return pl.kernel(
        _sc_gather_body,
        out_type=jax.ShapeDtypeStruct((_N,), jnp.float32),
        mesh=plsc.VectorSubcoreMesh(
            core_axis_name="c", subcore_axis_name="s",
            num_cores=_NC, num_subcores=_NS),
        scratch_types=[
            pltpu.VMEM((_NPW,), jnp.int32),
            pltpu.VMEM((_NPW,), jnp.float32),
            pltpu.VMEM((_NSP,), jnp.float32),
        ],
        compiler_params=pltpu.CompilerParams(needs_layout_passes=False),
    )

_BN = 8192


def _tc_body(x_ref, w_ref, o_ref):
    acc = jax.lax.dot_general(
        w_ref[...], x_ref[...], (((1,), (1,)), ((), ())),
        preferred_element_type=jnp.float32)
    o_ref[...] = jnp.reshape(acc, (_BN,))


def _tc_matvec(x, wrow):
    return pl.pallas_call(
        _tc_body,
        grid=(pl.cdiv(_N, _BN),),
        in_specs=[
            pl.BlockSpec((_BN, _D), lambda i: (i, 0)),
            pl.BlockSpec((1, _D), lambda i: (0, 0)),
        ],
        out_specs=pl.BlockSpec((_BN,), lambda i: (i,)),
        out_shape=jax.ShapeDtypeStruct((_N,), jnp.float32),
    )(x, wrow)


_BA = 65536


def _add_body(a_ref, b_ref, o_ref):
    o_ref[...] = a_ref[...] + b_ref[...]


def _tc_add(a, b):
    return pl.pallas_call(
        _add_body,
        grid=(pl.cdiv(_N, _BA),),
        in_specs=[
            pl.BlockSpec((_BA,), lambda i: (i,)),
            pl.BlockSpec((_BA,), lambda i: (i,)),
        ],
        out_specs=pl.BlockSpec((_BA,), lambda i: (i,)),
        out_shape=jax.ShapeDtypeStruct((_N,), jnp.float32),
    )(a, b)


def kernel(x, central_species, W, self_contributions):
    contrib = _sc_gather()(central_species, self_contributions)
    pred = _tc_matvec(x, W.reshape(1, _D))
    return _tc_add(pred, contrib)[:, None]


# speedup vs baseline: 1.1800x; 1.1800x over previous
import functools

import jax
import jax.numpy as jnp
from jax import lax
from jax.experimental import pallas as pl
from jax.experimental.pallas import tpu as pltpu
from jax.experimental.pallas import tpu_sc as plsc

_N = 500000
_D = 128
_NSP = 64

_BN = 16384
_BH = _BN // 2


def _tc_body(xa_ref, xb_ref, w_ref, o_ref):
    acc_a = jax.lax.dot_general(
        w_ref[...], xa_ref[...], (((1,), (1,)), ((), ())),
        preferred_element_type=jnp.float32)
    acc_b = jax.lax.dot_general(
        w_ref[...], xb_ref[...], (((1,), (1,)), ((), ())),
        preferred_element_type=jnp.float32)
    o_ref[pl.ds(0, _BH)] = jnp.reshape(acc_a, (_BH,))
    o_ref[pl.ds(_BH, _BH)] = jnp.reshape(acc_b, (_BH,))


def _tc_matvec(x, wrow):
    return pl.pallas_call(
        _tc_body,
        grid=(pl.cdiv(_N, _BN),),
        in_specs=[
            pl.BlockSpec((_BH, _D), lambda i: (2 * i, 0)),
            pl.BlockSpec((_BH, _D), lambda i: (2 * i + 1, 0)),
            pl.BlockSpec((1, _D), lambda i: (0, 0)),
        ],
        out_specs=pl.BlockSpec((_BN,), lambda i: (i,)),
        out_shape=jax.ShapeDtypeStruct((_N,), jnp.float32),
    )(x, x, wrow)


_BA = 131072


def _add_body(a_ref, b_ref, o_ref):
    o_ref[...] = a_ref[...] + b_ref[...]


def _tc_add(a, b):
    return pl.pallas_call(
        _add_body,
        grid=(pl.cdiv(_N, _BA),),
        in_specs=[
            pl.BlockSpec((_BA,), lambda i: (i,)),
            pl.BlockSpec((_BA,), lambda i: (i,)),
        ],
        out_specs=pl.BlockSpec((_BA,), lambda i: (i,)),
        out_shape=jax.ShapeDtypeStruct((_N,), jnp.float32),
    )(a, b)


_NC = 2
_NS = 16
_NW = _NC * _NS
_LANES = 16
_NPW = 15616
_TAIL_BASE = _NW * _NPW
_TAIL = _N - _TAIL_BASE


def _sc_gather_body(species_hbm, table_hbm, out_hbm, idx_v, val_v, tab_v):
    wid = lax.axis_index("s") * _NC + lax.axis_index("c")
    pltpu.sync_copy(table_hbm, tab_v)

    def run(base, n):
        pltpu.sync_copy(species_hbm.at[pl.ds(base, n)], idx_v.at[pl.ds(0, n)])

        @plsc.parallel_loop(0, n // _LANES, unroll=8)
        def _(i):
            sl = pl.ds(i * _LANES, _LANES)
            val_v[sl] = plsc.load_gather(tab_v, [idx_v[sl]])

        pltpu.sync_copy(val_v.at[pl.ds(0, n)], out_hbm.at[pl.ds(base, n)])

    run(wid * _NPW, _NPW)

    @pl.when(wid == 0)
    def _tail():
        run(_TAIL_BASE, _TAIL)


@functools.cache
def _sc_gather():
    return pl.kernel(
        _sc_gather_body,
        out_type=jax.ShapeDtypeStruct((_N,), jnp.float32),
        mesh=plsc.VectorSubcoreMesh(
            core_axis_name="c", subcore_axis_name="s",
            num_cores=_NC, num_subcores=_NS),
        scratch_types=[
            pltpu.VMEM((_NPW,), jnp.int32),
            pltpu.VMEM((_NPW,), jnp.float32),
            pltpu.VMEM((_NSP,), jnp.float32),
        ],
        compiler_params=pltpu.CompilerParams(needs_layout_passes=False),
    )


def kernel(x, central_species, W, self_contributions):
    contrib = _sc_gather()(central_species, self_contributions)
    pred = _tc_matvec(x, W.reshape(1, _D))
    return _tc_add(pred, contrib)[:, None]
